# Initial kernel scaffold; baseline (speedup 1.0000x reference)
import jax
import jax.numpy as jnp
from jax.experimental import pallas as pl


def kernel(hidden, mem_keys, mem_values, Wq, bq, Wo, bo, Wg, bg):
    raise NotImplementedError("write your pallas kernel here")


# speedup vs baseline: 2.6150x; 2.6150x over previous
import functools

import jax
import jax.numpy as jnp
from jax import lax
from jax.experimental import pallas as pl
from jax.experimental.pallas import tpu as pltpu

B, L, D = 2, 2048, 1024
KD = 128
N_ENTRIES = 100000
TOP_K = 32

NB = 2048
N_PAD = ((N_ENTRIES + NB - 1) // NB) * NB
CHUNK = 128
N_CHUNKS = N_PAD // CHUNK
QB = 256
BL = B * L
NEG = -1e30


def _qproj_body(h_ref, wq_ref, bq_ref, q_ref):
    q_ref[...] = (
        jnp.dot(h_ref[...], wq_ref[...].T, preferred_element_type=jnp.float32)
        + bq_ref[...]
    )


def _qproj(hidden_flat, Wq, bq):
    return pl.pallas_call(
        _qproj_body,
        grid=(BL // 512,),
        in_specs=[
            pl.BlockSpec((512, D), lambda i: (i, 0)),
            pl.BlockSpec((KD, D), lambda i: (0, 0)),
            pl.BlockSpec((1, KD), lambda i: (0, 0)),
        ],
        out_specs=pl.BlockSpec((512, KD), lambda i: (i, 0)),
        out_shape=jax.ShapeDtypeStruct((BL, KD), jnp.float32),
    )(hidden_flat, Wq, bq.reshape(1, KD))


def _sim_body(q_ref, k_ref, sim_ref, m_ref):
    nb = pl.program_id(1)
    sim = jnp.dot(q_ref[...], k_ref[...].T, preferred_element_type=jnp.float32)
    col = nb * NB + lax.broadcasted_iota(jnp.int32, (QB, NB), 1)
    sim = jnp.where(col < N_ENTRIES, sim, NEG)
    sim_ref[...] = sim
    m_ref[0] = jnp.max(sim.reshape(QB, NB // CHUNK, CHUNK), axis=2)


def _sim(q, keys_pad):
    return pl.pallas_call(
        _sim_body,
        grid=(BL // QB, N_PAD // NB),
        in_specs=[
            pl.BlockSpec((QB, KD), lambda i, j: (i, 0)),
            pl.BlockSpec((NB, KD), lambda i, j: (j, 0)),
        ],
        out_specs=[
            pl.BlockSpec((QB, NB), lambda i, j: (i, j)),
            pl.BlockSpec((1, QB, NB // CHUNK), lambda i, j: (j, i, 0)),
        ],
        out_shape=[
            jax.ShapeDtypeStruct((BL, N_PAD), jnp.float32),
            jax.ShapeDtypeStruct((N_PAD // NB, BL, NB // CHUNK), jnp.float32),
        ],
    )(q, keys_pad)


def _topchunk_body(m_ref, val_ref, idx_ref):
    cur = m_ref[...]
    rows = cur.shape[0]
    ids = lax.broadcasted_iota(jnp.int32, cur.shape, 1)
    for k in range(TOP_K):
        v = jnp.max(cur, axis=1, keepdims=True)
        hit = cur == v
        i = jnp.min(jnp.where(hit, ids, N_CHUNKS), axis=1, keepdims=True)
        val_ref[:, k] = v[:, 0]
        idx_ref[:, k] = i[:, 0]
        cur = jnp.where(ids == i, NEG, cur)


def _topchunks(M):
    QB2 = 512
    return pl.pallas_call(
        _topchunk_body,
        grid=(BL // QB2,),
        in_specs=[pl.BlockSpec((QB2, N_CHUNKS), lambda i: (i, 0))],
        out_specs=[
            pl.BlockSpec((QB2, TOP_K), lambda i: (i, 0)),
            pl.BlockSpec((QB2, TOP_K), lambda i: (i, 0)),
        ],
        out_shape=[
            jax.ShapeDtypeStruct((BL, TOP_K), jnp.float32),
            jax.ShapeDtypeStruct((BL, TOP_K), jnp.int32),
        ],
    )(M)


def _outproj_body(h_ref, mo_ref, wo_ref, bo_ref, wg1_ref, wg2_ref, bg_ref, o_ref):
    h = h_ref[...]
    mo2 = (
        jnp.dot(mo_ref[...], wo_ref[...].T, preferred_element_type=jnp.float32)
        + bo_ref[...]
    )
    logit = (
        jnp.sum(h * wg1_ref[...], axis=1, keepdims=True)
        + jnp.sum(mo2 * wg2_ref[...], axis=1, keepdims=True)
        + bg_ref[...]
    )
    gate = jax.nn.sigmoid(logit)
    o_ref[...] = h + gate * mo2


def _outproj(hidden_flat, mo, Wo, bo, Wg, bg):
    QB3 = 512
    return pl.pallas_call(
        _outproj_body,
        grid=(BL // QB3,),
        in_specs=[
            pl.BlockSpec((QB3, D), lambda i: (i, 0)),
            pl.BlockSpec((QB3, D), lambda i: (i, 0)),
            pl.BlockSpec((D, D), lambda i: (0, 0)),
            pl.BlockSpec((1, D), lambda i: (0, 0)),
            pl.BlockSpec((1, D), lambda i: (0, 0)),
            pl.BlockSpec((1, D), lambda i: (0, 0)),
            pl.BlockSpec((1, 1), lambda i: (0, 0)),
        ],
        out_specs=pl.BlockSpec((QB3, D), lambda i: (i, 0)),
        out_shape=jax.ShapeDtypeStruct((BL, D), jnp.float32),
    )(
        hidden_flat, mo, Wo, bo.reshape(1, D),
        Wg[:, :D], Wg[:, D:], bg.reshape(1, 1),
    )


def kernel(hidden, mem_keys, mem_values, Wq, bq, Wo, bo, Wg, bg):
    hidden_flat = hidden.reshape(BL, D)
    keys_pad = jnp.pad(mem_keys, ((0, N_PAD - N_ENTRIES), (0, 0)))

    q = _qproj(hidden_flat, Wq, bq)
    sim, M3 = _sim(q, keys_pad)
    M = jnp.transpose(M3, (1, 0, 2)).reshape(BL, N_CHUNKS)
    cvals, cidx = _topchunks(M)

    sim3 = sim.reshape(BL, N_CHUNKS, CHUNK)
    cand = jnp.take_along_axis(sim3, cidx[:, :, None], axis=1)
    candflat = cand.reshape(BL, TOP_K * CHUNK)
    tv, tp = jax.lax.top_k(candflat, TOP_K)
    gidx = (
        jnp.take_along_axis(cidx, tp // CHUNK, axis=1) * CHUNK + tp % CHUNK
    )
    w = jax.nn.softmax(tv, axis=-1)
    rows = jnp.take(mem_values, gidx, axis=0)
    mo = jnp.einsum("qk,qkd->qd", w, rows)

    out = _outproj(hidden_flat, mo, Wo, bo, Wg, bg)
    return out.reshape(B, L, D)


# speedup vs baseline: 13.4478x; 5.1426x over previous
import functools

import jax
import jax.numpy as jnp
from jax import lax
from jax.experimental import pallas as pl
from jax.experimental.pallas import tpu as pltpu
from jax.experimental.pallas import tpu_sc as plsc

B, L, D = 2, 2048, 1024
KD = 128
N_ENTRIES = 100000
TOP_K = 32

NB = 2048
N_PAD = ((N_ENTRIES + NB - 1) // NB) * NB
CHUNK = 128
N_CHUNKS = N_PAD // CHUNK
QB = 256
BL = B * L
NEG = -1e30


def _qproj_body(h_ref, wq_ref, bq_ref, q_ref):
    q_ref[...] = (
        jnp.dot(h_ref[...], wq_ref[...].T, preferred_element_type=jnp.float32)
        + bq_ref[...]
    )


def _qproj(hidden_flat, Wq, bq):
    bl = hidden_flat.shape[0]
    return pl.pallas_call(
        _qproj_body,
        grid=(bl // 512,),
        in_specs=[
            pl.BlockSpec((512, D), lambda i: (i, 0)),
            pl.BlockSpec((KD, D), lambda i: (0, 0)),
            pl.BlockSpec((1, KD), lambda i: (0, 0)),
        ],
        out_specs=pl.BlockSpec((512, KD), lambda i: (i, 0)),
        out_shape=jax.ShapeDtypeStruct((bl, KD), jnp.float32),
    )(hidden_flat, Wq, bq.reshape(1, KD))


def _sim_body(q_ref, k_ref, sim_ref, m_ref):
    nb = pl.program_id(0)
    sim = jnp.dot(q_ref[...], k_ref[...].T, preferred_element_type=jnp.float32)
    col = nb * NB + lax.broadcasted_iota(jnp.int32, (QB, NB), 1)
    sim = jnp.where(col < N_ENTRIES, sim, NEG)
    simb = sim.astype(jnp.bfloat16)
    sim_ref[...] = pltpu.bitcast(simb, jnp.int32)
    m_ref[0] = jnp.max(
        simb.astype(jnp.float32).reshape(QB, NB // CHUNK, CHUNK), axis=2
    )


def _sim(q, keys_pad):
    bl = q.shape[0]
    return pl.pallas_call(
        _sim_body,
        grid=(N_PAD // NB, bl // QB),
        in_specs=[
            pl.BlockSpec((QB, KD), lambda j, i: (i, 0)),
            pl.BlockSpec((NB, KD), lambda j, i: (j, 0)),
        ],
        out_specs=[
            pl.BlockSpec((QB // 2, NB), lambda j, i: (i, j)),
            pl.BlockSpec((1, QB, NB // CHUNK), lambda j, i: (j, i, 0)),
        ],
        out_shape=[
            jax.ShapeDtypeStruct((bl // 2, N_PAD), jnp.int32),
            jax.ShapeDtypeStruct((N_PAD // NB, bl, NB // CHUNK), jnp.float32),
        ],
    )(q, keys_pad)


def _topchunk_body(m_ref, val_ref, idx_ref):
    cur = m_ref[...]
    rows = cur.shape[0]
    ids = lax.broadcasted_iota(jnp.int32, cur.shape, 1)
    for k in range(TOP_K):
        v = jnp.max(cur, axis=1, keepdims=True)
        hit = cur == v
        i = jnp.min(jnp.where(hit, ids, N_CHUNKS), axis=1, keepdims=True)
        val_ref[:, k] = v[:, 0]
        idx_ref[:, k] = i[:, 0]
        cur = jnp.where(ids == i, NEG, cur)


def _topchunks(M):
    bl = M.shape[0]
    QB2 = 512
    return pl.pallas_call(
        _topchunk_body,
        grid=(bl // QB2,),
        in_specs=[pl.BlockSpec((QB2, N_CHUNKS), lambda i: (i, 0))],
        out_specs=[
            pl.BlockSpec((QB2, TOP_K), lambda i: (i, 0)),
            pl.BlockSpec((QB2, TOP_K), lambda i: (i, 0)),
        ],
        out_shape=[
            jax.ShapeDtypeStruct((bl, TOP_K), jnp.float32),
            jax.ShapeDtypeStruct((bl, TOP_K), jnp.int32),
        ],
    )(M)


def _outproj_body(h_ref, mo_ref, wo_ref, bo_ref, wg1_ref, wg2_ref, bg_ref, o_ref):
    h = h_ref[...]
    mo2 = (
        jnp.dot(mo_ref[...], wo_ref[...].T, preferred_element_type=jnp.float32)
        + bo_ref[...]
    )
    logit = (
        jnp.sum(h * wg1_ref[...], axis=1, keepdims=True)
        + jnp.sum(mo2 * wg2_ref[...], axis=1, keepdims=True)
        + bg_ref[...]
    )
    gate = jax.nn.sigmoid(logit)
    o_ref[...] = h + gate * mo2


def _outproj(hidden_flat, mo, Wo, bo, Wg, bg):
    bl = hidden_flat.shape[0]
    QB3 = 512
    return pl.pallas_call(
        _outproj_body,
        grid=(bl // QB3,),
        in_specs=[
            pl.BlockSpec((QB3, D), lambda i: (i, 0)),
            pl.BlockSpec((QB3, D), lambda i: (i, 0)),
            pl.BlockSpec((D, D), lambda i: (0, 0)),
            pl.BlockSpec((1, D), lambda i: (0, 0)),
            pl.BlockSpec((1, D), lambda i: (0, 0)),
            pl.BlockSpec((1, D), lambda i: (0, 0)),
            pl.BlockSpec((1, 1), lambda i: (0, 0)),
        ],
        out_specs=pl.BlockSpec((QB3, D), lambda i: (i, 0)),
        out_shape=jax.ShapeDtypeStruct((bl, D), jnp.float32),
    )(
        hidden_flat, mo, Wo, bo.reshape(1, D),
        Wg[:, :D], Wg[:, D:], bg.reshape(1, 1),
    )


NW = 32
LANE = 16


def _merge16(hi, ihi, lo, ilo, v, iv):
    sv, si = plsc.sort_key_val(v, iv, descending=True)
    rsv = lax.rev(sv, (0,))
    rsi = lax.rev(si, (0,))
    take = hi >= rsv
    h1 = jnp.where(take, hi, rsv)
    ih1 = jnp.where(take, ihi, rsi)
    r1 = jnp.where(take, rsv, hi)
    ir1 = jnp.where(take, rsi, ihi)
    hi2, ihi2 = plsc.sort_key_val(h1, ih1, descending=True)
    r1s, ir1s = plsc.sort_key_val(r1, ir1, descending=True)
    rr = lax.rev(r1s, (0,))
    irr = lax.rev(ir1s, (0,))
    take2 = lo >= rr
    l1 = jnp.where(take2, lo, rr)
    il1 = jnp.where(take2, ilo, irr)
    lo2, ilo2 = plsc.sort_key_val(l1, il1, descending=True)
    return hi2, ihi2, lo2, ilo2


def _sc_body(QPW, simrows, cidx, cvals, values, out,
             cidx_v, cvals_v, gidx_v, cand_v, colv, coli,
             vidx_v, w_v, rows_v, acc_v, sem_c, sem_v0, sem_v1):
    wid = lax.axis_index("s") * 2 + lax.axis_index("c")
    q0 = wid * QPW
    pltpu.sync_copy(cidx.at[pl.ds(q0, QPW)], cidx_v)
    pltpu.sync_copy(cvals.at[pl.ds(q0, QPW)], cvals_v)
    iota = lax.broadcasted_iota(jnp.int32, (LANE,), 0)
    sem_v = (sem_v0, sem_v1)

    def _start_cand(i, b):
        q = q0 + i
        base = jnp.full((LANE,), (q >> 1) * N_CHUNKS, jnp.int32)
        gidx_v[b, pl.ds(0, LANE)] = cidx_v[i, pl.ds(0, LANE)] + base
        gidx_v[b, pl.ds(LANE, LANE)] = cidx_v[i, pl.ds(LANE, LANE)] + base
        pltpu.async_copy(simrows.at[gidx_v.at[b]], cand_v.at[b], sem_c)

    def _wait_cand(b):
        pltpu.make_async_copy(
            simrows.at[gidx_v.at[b]], cand_v.at[b], sem_c
        ).wait()

    def _select(i, b):
        q = q0 + i
        t = -plsc.cummax(-cvals_v[i, pl.ds(LANE, LANE)])[LANE - 1]
        tv = jnp.full((LANE,), t)

        sh = (q & 1) * 16

        def _chunk(k, cnt):
            for jv in range(CHUNK // LANE):
                w = cand_v[b, k, pl.ds(jv * LANE, LANE)]
                v = plsc.bitcast(lax.shift_right_logical(w, sh) << 16,
                                 jnp.float32)
                idv = iota + (k * CHUNK + jv * LANE)
                m = v >= tv
                plsc.store_compressed(colv.at[pl.ds(cnt, LANE)], v, mask=m)
                plsc.store_compressed(coli.at[pl.ds(cnt, LANE)], idv, mask=m)
                cnt = cnt + plsc.cumsum(m.astype(jnp.int32))[LANE - 1]
            return cnt

        cnt = pl.loop(0, TOP_K, init_carry=jnp.int32(0))(_chunk)
        colv[pl.ds(cnt, LANE)] = jnp.full((LANE,), NEG, jnp.float32)
        coli[pl.ds(cnt, LANE)] = jnp.zeros((LANE,), jnp.int32)

        ng = (cnt + LANE - 1) // LANE
        neg = jnp.full((LANE,), NEG, jnp.float32)
        zero = jnp.zeros((LANE,), jnp.int32)

        def _group(g, carry):
            return _merge16(*carry, colv[pl.ds(g * LANE, LANE)],
                            coli[pl.ds(g * LANE, LANE)])

        hi, ihi, lo, ilo = pl.loop(0, ng, init_carry=(neg, zero, neg, zero))(
            _group
        )

        mx = hi[0]
        eh = jnp.exp(hi - mx)
        el = jnp.exp(lo - mx)
        s = plsc.cumsum(eh)[LANE - 1] + plsc.cumsum(el)[LANE - 1]
        w_v[b, pl.ds(0, LANE)] = eh / s
        w_v[b, pl.ds(LANE, LANE)] = el / s

        qv = jnp.full((LANE,), i, jnp.int32)
        ch = plsc.load_gather(cidx_v, [qv, lax.shift_right_logical(ihi, 7)])
        cl = plsc.load_gather(cidx_v, [qv, lax.shift_right_logical(ilo, 7)])
        vidx_v[b, pl.ds(0, LANE)] = ch * CHUNK + (ihi & (CHUNK - 1))
        vidx_v[b, pl.ds(LANE, LANE)] = cl * CHUNK + (ilo & (CHUNK - 1))

    def _wsum_out(i, b):
        pltpu.make_async_copy(
            values.at[vidx_v.at[b]], rows_v.at[b], sem_v[b]
        ).wait()
        wv0 = w_v[b, pl.ds(0, LANE)]
        wv1 = w_v[b, pl.ds(LANE, LANE)]
        wks = [wv0[k] for k in range(LANE)] + [wv1[k] for k in range(LANE)]

        @pl.loop(0, D // LANE)
        def _dim(j):
            acc = jnp.zeros((LANE,), jnp.float32)
            for k in range(TOP_K):
                acc = acc + wks[k] * rows_v[b, k, pl.ds(j * LANE, LANE)]
            acc_v[pl.ds(j * LANE, LANE)] = acc

        pltpu.sync_copy(acc_v, out.at[q0 + i])

    _start_cand(jnp.int32(0), 0)

    @pl.loop(0, QPW, step=2)
    def _pair(i0):
        for b in (0, 1):
            i = i0 + b
            _wait_cand(b)

            @pl.when(i + 1 < QPW)
            def _():
                _start_cand(i + 1, 1 - b)

            _select(i, b)
            pltpu.async_copy(values.at[vidx_v.at[b]], rows_v.at[b], sem_v[b])

            @pl.when(i > 0)
            def _():
                _wsum_out(i - 1, 1 - b)

    _wsum_out(jnp.int32(QPW - 1), 1)


def _sc_select_gather(sim, cidx, cvals, mem_values):
    bl = cidx.shape[0]
    QPW = bl // NW
    simrows = sim.reshape(bl // 2 * N_CHUNKS, CHUNK)
    f = pl.kernel(
        functools.partial(_sc_body, QPW),
        out_type=jax.ShapeDtypeStruct((bl, D), jnp.float32),
        mesh=plsc.VectorSubcoreMesh(core_axis_name="c", subcore_axis_name="s"),
        compiler_params=pltpu.CompilerParams(needs_layout_passes=False),
        scratch_types=[
            pltpu.VMEM((QPW, TOP_K), jnp.int32),
            pltpu.VMEM((QPW, TOP_K), jnp.float32),
            pltpu.VMEM((2, TOP_K), jnp.int32),
            pltpu.VMEM((2, TOP_K, CHUNK), jnp.int32),
            pltpu.VMEM((TOP_K * CHUNK + LANE,), jnp.float32),
            pltpu.VMEM((TOP_K * CHUNK + LANE,), jnp.int32),
            pltpu.VMEM((2, TOP_K), jnp.int32),
            pltpu.VMEM((2, TOP_K), jnp.float32),
            pltpu.VMEM((2, TOP_K, D), jnp.float32),
            pltpu.VMEM((D,), jnp.float32),
            pltpu.SemaphoreType.DMA,
            pltpu.SemaphoreType.DMA,
            pltpu.SemaphoreType.DMA,
        ],
    )
    return f(simrows, cidx, cvals, mem_values)


def kernel(hidden, mem_keys, mem_values, Wq, bq, Wo, bo, Wg, bg):
    hidden_flat = hidden.reshape(BL, D)
    keys_pad = jnp.pad(mem_keys, ((0, N_PAD - N_ENTRIES), (0, 0)))
    keys_bf = keys_pad.astype(jnp.bfloat16)

    HALF = BL // 2
    outs = []
    for h in range(2):
        hf = lax.dynamic_slice_in_dim(hidden_flat, h * HALF, HALF)
        q = _qproj(hf, Wq, bq)
        sim, M3 = _sim(q.astype(jnp.bfloat16), keys_bf)
        M = jnp.transpose(M3, (1, 0, 2)).reshape(HALF, N_CHUNKS)
        cvals, cidx = _topchunks(M)
        mo = _sc_select_gather(sim, cidx, cvals, mem_values)
        outs.append(_outproj(hf, mo, Wo, bo, Wg, bg))
    return jnp.concatenate(outs, axis=0).reshape(B, L, D)


# speedup vs baseline: 14.4791x; 1.0767x over previous
import functools

import jax
import jax.numpy as jnp
from jax import lax
from jax.experimental import pallas as pl
from jax.experimental.pallas import tpu as pltpu
from jax.experimental.pallas import tpu_sc as plsc

B, L, D = 2, 2048, 1024
KD = 128
N_ENTRIES = 100000
TOP_K = 32

NB = 2048
N_PAD = ((N_ENTRIES + NB - 1) // NB) * NB
CHUNK = 128
N_CHUNKS = N_PAD // CHUNK
QB = 256
BL = B * L
NEG = -1e30


def _qproj_body(h_ref, wq_ref, bq_ref, q_ref):
    q_ref[...] = (
        jnp.dot(h_ref[...], wq_ref[...].T, preferred_element_type=jnp.float32)
        + bq_ref[...]
    )


def _qproj(hidden_flat, Wq, bq):
    bl = hidden_flat.shape[0]
    return pl.pallas_call(
        _qproj_body,
        grid=(bl // 512,),
        in_specs=[
            pl.BlockSpec((512, D), lambda i: (i, 0)),
            pl.BlockSpec((KD, D), lambda i: (0, 0)),
            pl.BlockSpec((1, KD), lambda i: (0, 0)),
        ],
        out_specs=pl.BlockSpec((512, KD), lambda i: (i, 0)),
        out_shape=jax.ShapeDtypeStruct((bl, KD), jnp.float32),
    )(hidden_flat, Wq, bq.reshape(1, KD))


def _sim_body(q_ref, k_ref, sim_ref, m_ref):
    nb = pl.program_id(0)
    sim = jnp.dot(q_ref[...], k_ref[...].T, preferred_element_type=jnp.float32)
    col = nb * NB + lax.broadcasted_iota(jnp.int32, (QB, NB), 1)
    sim = jnp.where(col < N_ENTRIES, sim, NEG)
    simb = sim.astype(jnp.bfloat16)
    sim_ref[...] = pltpu.bitcast(simb, jnp.int32)
    m_ref[0] = jnp.max(
        simb.astype(jnp.float32).reshape(QB, NB // CHUNK, CHUNK), axis=2
    )


def _sim(q, keys_pad):
    bl = q.shape[0]
    return pl.pallas_call(
        _sim_body,
        grid=(N_PAD // NB, bl // QB),
        in_specs=[
            pl.BlockSpec((QB, KD), lambda j, i: (i, 0)),
            pl.BlockSpec((NB, KD), lambda j, i: (j, 0)),
        ],
        out_specs=[
            pl.BlockSpec((QB // 2, NB), lambda j, i: (i, j)),
            pl.BlockSpec((1, QB, NB // CHUNK), lambda j, i: (j, i, 0)),
        ],
        out_shape=[
            jax.ShapeDtypeStruct((bl // 2, N_PAD), jnp.int32),
            jax.ShapeDtypeStruct((N_PAD // NB, bl, NB // CHUNK), jnp.float32),
        ],
    )(q, keys_pad)


def _topchunk_body(m_ref, val_ref, idx_ref):
    cur = m_ref[...]
    rows = cur.shape[0]
    ids = lax.broadcasted_iota(jnp.int32, cur.shape, 1)
    for k in range(TOP_K):
        v = jnp.max(cur, axis=1, keepdims=True)
        hit = cur == v
        i = jnp.min(jnp.where(hit, ids, N_CHUNKS), axis=1, keepdims=True)
        val_ref[:, k] = v[:, 0]
        idx_ref[:, k] = i[:, 0]
        cur = jnp.where(ids == i, NEG, cur)


def _topchunks(M):
    bl = M.shape[0]
    QB2 = 512
    return pl.pallas_call(
        _topchunk_body,
        grid=(bl // QB2,),
        in_specs=[pl.BlockSpec((QB2, N_CHUNKS), lambda i: (i, 0))],
        out_specs=[
            pl.BlockSpec((QB2, TOP_K), lambda i: (i, 0)),
            pl.BlockSpec((QB2, TOP_K), lambda i: (i, 0)),
        ],
        out_shape=[
            jax.ShapeDtypeStruct((bl, TOP_K), jnp.float32),
            jax.ShapeDtypeStruct((bl, TOP_K), jnp.int32),
        ],
    )(M)


def _outproj_body(h_ref, mo_ref, wo_ref, bo_ref, wg1_ref, wg2_ref, bg_ref, o_ref):
    h = h_ref[...]
    mo2 = (
        jnp.dot(mo_ref[...], wo_ref[...].T, preferred_element_type=jnp.float32)
        + bo_ref[...]
    )
    logit = (
        jnp.sum(h * wg1_ref[...], axis=1, keepdims=True)
        + jnp.sum(mo2 * wg2_ref[...], axis=1, keepdims=True)
        + bg_ref[...]
    )
    gate = jax.nn.sigmoid(logit)
    o_ref[...] = h + gate * mo2


def _outproj(hidden_flat, mo, Wo, bo, Wg, bg):
    bl = hidden_flat.shape[0]
    QB3 = 512
    return pl.pallas_call(
        _outproj_body,
        grid=(bl // QB3,),
        in_specs=[
            pl.BlockSpec((QB3, D), lambda i: (i, 0)),
            pl.BlockSpec((QB3, D), lambda i: (i, 0)),
            pl.BlockSpec((D, D), lambda i: (0, 0)),
            pl.BlockSpec((1, D), lambda i: (0, 0)),
            pl.BlockSpec((1, D), lambda i: (0, 0)),
            pl.BlockSpec((1, D), lambda i: (0, 0)),
            pl.BlockSpec((1, 1), lambda i: (0, 0)),
        ],
        out_specs=pl.BlockSpec((QB3, D), lambda i: (i, 0)),
        out_shape=jax.ShapeDtypeStruct((bl, D), jnp.float32),
    )(
        hidden_flat, mo, Wo, bo.reshape(1, D),
        Wg[:, :D], Wg[:, D:], bg.reshape(1, 1),
    )


NW = 32
LANE = 16


def _merge16(hi, ihi, lo, ilo, v, iv):
    sv, si = plsc.sort_key_val(v, iv, descending=True)
    rsv = lax.rev(sv, (0,))
    rsi = lax.rev(si, (0,))
    take = hi >= rsv
    h1 = jnp.where(take, hi, rsv)
    ih1 = jnp.where(take, ihi, rsi)
    r1 = jnp.where(take, rsv, hi)
    ir1 = jnp.where(take, rsi, ihi)
    hi2, ihi2 = plsc.sort_key_val(h1, ih1, descending=True)
    r1s, ir1s = plsc.sort_key_val(r1, ir1, descending=True)
    rr = lax.rev(r1s, (0,))
    irr = lax.rev(ir1s, (0,))
    take2 = lo >= rr
    l1 = jnp.where(take2, lo, rr)
    il1 = jnp.where(take2, ilo, irr)
    lo2, ilo2 = plsc.sort_key_val(l1, il1, descending=True)
    return hi2, ihi2, lo2, ilo2


def _sc_body(QPW, simrows, cidx, cvals, values, out,
             cidx_v, cvals_v, gidx_v, cand_v, colv, coli,
             vidx_v, w_v, rows_v, acc_v, sem_c, sem_v0, sem_v1):
    wid = lax.axis_index("s") * 2 + lax.axis_index("c")
    q0 = wid * QPW
    pltpu.sync_copy(cidx.at[pl.ds(q0, QPW)], cidx_v)
    pltpu.sync_copy(cvals.at[pl.ds(q0, QPW)], cvals_v)
    iota = lax.broadcasted_iota(jnp.int32, (LANE,), 0)
    sem_v = (sem_v0, sem_v1)

    def _start_cand(i, b):
        q = q0 + i
        base = jnp.full((LANE,), (q >> 1) * N_CHUNKS, jnp.int32)
        gidx_v[b, pl.ds(0, LANE)] = cidx_v[i, pl.ds(0, LANE)] + base
        gidx_v[b, pl.ds(LANE, LANE)] = cidx_v[i, pl.ds(LANE, LANE)] + base
        pltpu.async_copy(simrows.at[gidx_v.at[b]], cand_v.at[b], sem_c)

    def _wait_cand(b):
        pltpu.make_async_copy(
            simrows.at[gidx_v.at[b]], cand_v.at[b], sem_c
        ).wait()

    def _select(i, b):
        q = q0 + i
        t = -plsc.cummax(-cvals_v[i, pl.ds(LANE, LANE)])[LANE - 1]
        tv = jnp.full((LANE,), t)

        sh = (q & 1) * 16

        def _chunk(k, cnt):
            for jv in range(CHUNK // LANE):
                w = cand_v[b, k, pl.ds(jv * LANE, LANE)]
                v = plsc.bitcast(lax.shift_right_logical(w, sh) << 16,
                                 jnp.float32)
                idv = iota + (k * CHUNK + jv * LANE)
                m = v >= tv
                plsc.store_compressed(colv.at[pl.ds(cnt, LANE)], v, mask=m)
                plsc.store_compressed(coli.at[pl.ds(cnt, LANE)], idv, mask=m)
                cnt = cnt + plsc.cumsum(m.astype(jnp.int32))[LANE - 1]
            return cnt

        cnt = pl.loop(0, TOP_K, init_carry=jnp.int32(0))(_chunk)
        colv[pl.ds(cnt, LANE)] = jnp.full((LANE,), NEG, jnp.float32)
        coli[pl.ds(cnt, LANE)] = jnp.zeros((LANE,), jnp.int32)

        ng = (cnt + LANE - 1) // LANE
        neg = jnp.full((LANE,), NEG, jnp.float32)
        zero = jnp.zeros((LANE,), jnp.int32)

        def _group(g, carry):
            return _merge16(*carry, colv[pl.ds(g * LANE, LANE)],
                            coli[pl.ds(g * LANE, LANE)])

        hi, ihi, lo, ilo = pl.loop(0, ng, init_carry=(neg, zero, neg, zero))(
            _group
        )

        mx = hi[0]
        eh = jnp.exp(hi - mx)
        el = jnp.exp(lo - mx)
        s = plsc.cumsum(eh)[LANE - 1] + plsc.cumsum(el)[LANE - 1]
        w_v[b, pl.ds(0, LANE)] = eh / s
        w_v[b, pl.ds(LANE, LANE)] = el / s

        qv = jnp.full((LANE,), i, jnp.int32)
        ch = plsc.load_gather(cidx_v, [qv, lax.shift_right_logical(ihi, 7)])
        cl = plsc.load_gather(cidx_v, [qv, lax.shift_right_logical(ilo, 7)])
        vidx_v[b, pl.ds(0, LANE)] = ch * CHUNK + (ihi & (CHUNK - 1))
        vidx_v[b, pl.ds(LANE, LANE)] = cl * CHUNK + (ilo & (CHUNK - 1))

    def _wsum_out(i, b):
        pltpu.make_async_copy(
            values.at[vidx_v.at[b]], rows_v.at[b], sem_v[b]
        ).wait()
        wv0 = w_v[b, pl.ds(0, LANE)]
        wv1 = w_v[b, pl.ds(LANE, LANE)]
        wks = [wv0[k] for k in range(LANE)] + [wv1[k] for k in range(LANE)]

        @pl.loop(0, D // LANE)
        def _dim(j):
            acc = jnp.zeros((LANE,), jnp.float32)
            for k in range(TOP_K):
                acc = acc + wks[k] * rows_v[b, k, pl.ds(j * LANE, LANE)]
            acc_v[pl.ds(j * LANE, LANE)] = acc

        pltpu.sync_copy(acc_v, out.at[q0 + i])

    _start_cand(jnp.int32(0), 0)

    @pl.loop(0, QPW, step=2)
    def _pair(i0):
        for b in (0, 1):
            i = i0 + b
            _wait_cand(b)

            @pl.when(i + 1 < QPW)
            def _():
                _start_cand(i + 1, 1 - b)

            _select(i, b)
            pltpu.async_copy(values.at[vidx_v.at[b]], rows_v.at[b], sem_v[b])

            @pl.when(i > 0)
            def _():
                _wsum_out(i - 1, 1 - b)

    _wsum_out(jnp.int32(QPW - 1), 1)


def _sc_select_gather(sim, cidx, cvals, mem_values):
    bl = cidx.shape[0]
    QPW = bl // NW
    simrows = sim.reshape(bl // 2 * N_CHUNKS, CHUNK)
    f = pl.kernel(
        functools.partial(_sc_body, QPW),
        out_type=jax.ShapeDtypeStruct((bl, D), jnp.float32),
        mesh=plsc.VectorSubcoreMesh(core_axis_name="c", subcore_axis_name="s"),
        compiler_params=pltpu.CompilerParams(needs_layout_passes=False),
        scratch_types=[
            pltpu.VMEM((QPW, TOP_K), jnp.int32),
            pltpu.VMEM((QPW, TOP_K), jnp.float32),
            pltpu.VMEM((2, TOP_K), jnp.int32),
            pltpu.VMEM((2, TOP_K, CHUNK), jnp.int32),
            pltpu.VMEM((TOP_K * CHUNK + LANE,), jnp.float32),
            pltpu.VMEM((TOP_K * CHUNK + LANE,), jnp.int32),
            pltpu.VMEM((2, TOP_K), jnp.int32),
            pltpu.VMEM((2, TOP_K), jnp.float32),
            pltpu.VMEM((2, TOP_K, D), jnp.float32),
            pltpu.VMEM((D,), jnp.float32),
            pltpu.SemaphoreType.DMA,
            pltpu.SemaphoreType.DMA,
            pltpu.SemaphoreType.DMA,
        ],
    )
    return f(simrows, cidx, cvals, mem_values)


def kernel(hidden, mem_keys, mem_values, Wq, bq, Wo, bo, Wg, bg):
    hidden_flat = hidden.reshape(BL, D)
    keys_pad = jnp.pad(mem_keys, ((0, N_PAD - N_ENTRIES), (0, 0)))
    keys_bf = keys_pad.astype(jnp.bfloat16)

    NSLICE = 4
    HALF = BL // NSLICE
    outs = []
    for h in range(NSLICE):
        hf = lax.dynamic_slice_in_dim(hidden_flat, h * HALF, HALF)
        q = _qproj(hf, Wq, bq)
        sim, M3 = _sim(q.astype(jnp.bfloat16), keys_bf)
        M = jnp.transpose(M3, (1, 0, 2)).reshape(HALF, N_CHUNKS)
        cvals, cidx = _topchunks(M)
        mo = _sc_select_gather(sim, cidx, cvals, mem_values)
        outs.append(_outproj(hf, mo, Wo, bo, Wg, bg))
    return jnp.concatenate(outs, axis=0).reshape(B, L, D)
